# baseline (speedup 1.0000x reference)
import functools
import math

import jax
import jax.numpy as jnp
from jax import lax
from jax.experimental import pallas as pl
from jax.experimental.pallas import tpu as pltpu
from jax.experimental.pallas import tpu_sc as plsc

D_MODEL = 1024
SCALE = math.sqrt(D_MODEL)

_info = plsc.get_sparse_core_info()
_NC = _info.num_cores
_NS = _info.num_subcores
_LANES = _info.num_lanes
_NW = _NC * _NS


@functools.lru_cache(maxsize=None)
def _make_gather(B, V, D, chunk):
    b_per_w = B // _NW
    n_chunks = b_per_w // chunk
    vregs_per_row = D // _LANES
    mesh = plsc.VectorSubcoreMesh(core_axis_name="c", subcore_axis_name="s")

    @functools.partial(
        pl.kernel,
        out_type=jax.ShapeDtypeStruct((B, D), jnp.float32),
        mesh=mesh,
        scratch_types=[
            pltpu.VMEM((b_per_w,), jnp.int32),
            pltpu.VMEM((chunk, D), jnp.float32),
            pltpu.SemaphoreType.DMA,
        ],
    )
    def k(tok_hbm, table_hbm, out_hbm, idx_v, rows_v, sem):
        wid = lax.axis_index("s") * _NC + lax.axis_index("c")
        base = wid * b_per_w
        pltpu.sync_copy(tok_hbm.at[pl.ds(base, b_per_w)], idx_v)

        def body(q, _):
            off = pl.multiple_of(q * chunk, chunk)
            pltpu.async_copy(
                table_hbm.at[idx_v.at[pl.ds(off, chunk)]], rows_v, sem
            ).wait()

            def scale_row(r, _):
                for c in range(vregs_per_row):
                    sl = pl.ds(c * _LANES, _LANES)
                    rows_v[r, sl] = rows_v[r, sl] * SCALE
                return 0

            lax.fori_loop(0, chunk, scale_row, 0)
            pltpu.sync_copy(rows_v, out_hbm.at[pl.ds(base + off, chunk)])
            return 0

        lax.fori_loop(0, n_chunks, body, 0)

    return k


def kernel(token, embedding):
    b, s = token.shape
    V, D = embedding.shape
    tok = token.reshape(b * s).astype(jnp.int32)
    out = _make_gather(b * s, V, D, 32)(tok, embedding)
    return out.reshape(b, s, D)


# speedup vs baseline: 1.5417x; 1.5417x over previous
import functools
import math

import jax
import jax.numpy as jnp
from jax import lax
from jax.experimental import pallas as pl
from jax.experimental.pallas import tpu as pltpu
from jax.experimental.pallas import tpu_sc as plsc

D_MODEL = 1024
SCALE = math.sqrt(D_MODEL)

_info = plsc.get_sparse_core_info()
_NC = _info.num_cores
_NS = _info.num_subcores
_LANES = _info.num_lanes
_NW = _NC * _NS


@functools.lru_cache(maxsize=None)
def _make_gather(B, V, D, chunk):
    b_per_w = B // _NW
    n_chunks = b_per_w // chunk
    assert n_chunks % 2 == 0
    vregs_per_row = D // _LANES
    mesh = plsc.VectorSubcoreMesh(core_axis_name="c", subcore_axis_name="s")

    @functools.partial(
        pl.kernel,
        out_type=jax.ShapeDtypeStruct((B, D), jnp.float32),
        mesh=mesh,
        scratch_types=[
            pltpu.VMEM((b_per_w,), jnp.int32),
            pltpu.VMEM((chunk, D), jnp.float32),
            pltpu.VMEM((chunk, D), jnp.float32),
            pltpu.VMEM((chunk, D), jnp.float32),
            pltpu.VMEM((chunk, D), jnp.float32),
            pltpu.SemaphoreType.DMA,
            pltpu.SemaphoreType.DMA,
            pltpu.SemaphoreType.DMA,
            pltpu.SemaphoreType.DMA,
        ],
    )
    def k(tok_hbm, table_hbm, out_hbm, idx_v, g0, g1, s0, s1,
          gsem0, gsem1, ssem0, ssem1):
        wid = lax.axis_index("s") * _NC + lax.axis_index("c")
        base = wid * b_per_w
        pltpu.sync_copy(tok_hbm.at[pl.ds(base, b_per_w)], idx_v)

        def idx_slice(off):
            return idx_v.at[pl.ds(pl.multiple_of(off, 8), chunk)]

        def gather_wait(gbuf, gsem):
            pltpu.make_async_copy(table_hbm.at[pl.ds(0, chunk)], gbuf, gsem).wait()

        def store_wait(sbuf, ssem):
            pltpu.make_async_copy(sbuf, out_hbm.at[pl.ds(0, chunk)], ssem).wait()

        def scale_chunk(gbuf, sbuf):
            def row(r, _):
                for c in range(vregs_per_row):
                    sl = pl.ds(c * _LANES, _LANES)
                    sbuf[r, sl] = gbuf[r, sl] * SCALE
                return 0
            lax.fori_loop(0, chunk, row, 0)

        pltpu.async_copy(table_hbm.at[idx_slice(0)], g0, gsem0)
        pltpu.async_copy(table_hbm.at[idx_slice(chunk)], g1, gsem1)

        def body(i, _):
            for half, (gbuf, gsem, sbuf, ssem) in enumerate(
                    ((g0, gsem0, s0, ssem0), (g1, gsem1, s1, ssem1))):
                q = 2 * i + half
                off = pl.multiple_of(q * chunk, 8)
                gather_wait(gbuf, gsem)

                @pl.when(i > 0)
                def _():
                    store_wait(sbuf, ssem)

                scale_chunk(gbuf, sbuf)
                pltpu.async_copy(sbuf, out_hbm.at[pl.ds(base + off, chunk)], ssem)

                @pl.when(q + 2 < n_chunks)
                def _():
                    noff = pl.multiple_of((q + 2) * chunk, 8)
                    pltpu.async_copy(table_hbm.at[idx_slice(noff)], gbuf, gsem)
            return 0

        lax.fori_loop(0, n_chunks // 2, body, 0)
        store_wait(s0, ssem0)
        store_wait(s1, ssem1)

    return k


def kernel(token, embedding):
    b, s = token.shape
    V, D = embedding.shape
    tok = token.reshape(b * s).astype(jnp.int32)
    out = _make_gather(b * s, V, D, 16)(tok, embedding)
    return out.reshape(b, s, D)


# speedup vs baseline: 1.6192x; 1.0502x over previous
import functools
import math

import jax
import jax.numpy as jnp
from jax import lax
from jax.experimental import pallas as pl
from jax.experimental.pallas import tpu as pltpu
from jax.experimental.pallas import tpu_sc as plsc

D_MODEL = 1024
SCALE = math.sqrt(D_MODEL)

_info = plsc.get_sparse_core_info()
_NC = _info.num_cores
_NS = _info.num_subcores
_LANES = _info.num_lanes
_NW = _NC * _NS

_NG = 4
_NS_BUF = 2


@functools.lru_cache(maxsize=None)
def _make_gather(B, V, D, chunk):
    b_per_w = B // _NW
    n_chunks = b_per_w // chunk
    assert n_chunks % _NG == 0
    vregs_per_row = D // _LANES
    mesh = plsc.VectorSubcoreMesh(core_axis_name="c", subcore_axis_name="s")

    @functools.partial(
        pl.kernel,
        out_type=jax.ShapeDtypeStruct((B, D), jnp.float32),
        mesh=mesh,
        scratch_types=(
            [pltpu.VMEM((b_per_w,), jnp.int32)]
            + [pltpu.VMEM((chunk, D), jnp.float32)] * (_NG + _NS_BUF)
            + [pltpu.SemaphoreType.DMA] * (_NG + _NS_BUF)
        ),
    )
    def k(tok_hbm, table_hbm, out_hbm, idx_v, *bufs_and_sems):
        g = bufs_and_sems[:_NG]
        s = bufs_and_sems[_NG:_NG + _NS_BUF]
        gsem = bufs_and_sems[_NG + _NS_BUF:2 * _NG + _NS_BUF]
        ssem = bufs_and_sems[2 * _NG + _NS_BUF:]

        wid = lax.axis_index("s") * _NC + lax.axis_index("c")
        base = wid * b_per_w
        pltpu.sync_copy(tok_hbm.at[pl.ds(base, b_per_w)], idx_v)

        def idx_slice(off):
            return idx_v.at[pl.ds(pl.multiple_of(off, 8), chunk)]

        def gather_wait(j):
            pltpu.make_async_copy(
                table_hbm.at[pl.ds(0, chunk)], g[j], gsem[j]).wait()

        def store_wait(j):
            pltpu.make_async_copy(
                s[j], out_hbm.at[pl.ds(0, chunk)], ssem[j]).wait()

        def scale_chunk(gbuf, sbuf):
            def row(r, _):
                for c in range(vregs_per_row):
                    sl = pl.ds(c * _LANES, _LANES)
                    sbuf[r, sl] = gbuf[r, sl] * SCALE
                return 0
            lax.fori_loop(0, chunk, row, 0)

        for j in range(_NG):
            pltpu.async_copy(table_hbm.at[idx_slice(j * chunk)], g[j], gsem[j])

        def body(i, _):
            for half in range(_NG):
                q = _NG * i + half
                sj = half % _NS_BUF
                off = pl.multiple_of(q * chunk, 8)
                gather_wait(half)
                if half >= _NS_BUF:
                    store_wait(sj)
                else:
                    @pl.when(i > 0)
                    def _():
                        store_wait(sj)
                scale_chunk(g[half], s[sj])
                pltpu.async_copy(
                    s[sj], out_hbm.at[pl.ds(base + off, chunk)], ssem[sj])

                @pl.when(q + _NG < n_chunks)
                def _():
                    noff = pl.multiple_of((q + _NG) * chunk, 8)
                    pltpu.async_copy(
                        table_hbm.at[idx_slice(noff)], g[half], gsem[half])
            return 0

        lax.fori_loop(0, n_chunks // _NG, body, 0)
        for j in range(_NS_BUF):
            store_wait(j)

    return k


def kernel(token, embedding):
    b, s = token.shape
    V, D = embedding.shape
    tok = token.reshape(b * s).astype(jnp.int32)
    out = _make_gather(b * s, V, D, 16)(tok, embedding)
    return out.reshape(b, s, D)
